# baseline (speedup 1.0000x reference)
import functools

import jax
import jax.numpy as jnp
from jax import lax
from jax.experimental import pallas as pl
from jax.experimental.pallas import tpu as pltpu
from jax.experimental.pallas import tpu_sc as plsc

NC = 2
NS = 16
NW = NC * NS

CHUNK = 128
NBUF = 4


def _gather_body(n_chunks, n_j, ids_hbm, table_hbm, out_hbm, idx_v, rows_v,
                 trans_v, in_sems, out_sems):
  wid = lax.axis_index("s") * NC + lax.axis_index("c")
  chunk0 = wid * n_chunks

  pltpu.sync_copy(ids_hbm.at[pl.ds(chunk0, n_chunks)], idx_v)

  def start_gather(g, b):
    pltpu.async_copy(table_hbm.at[idx_v.at[g]], rows_v.at[b],
                     in_sems.at[b])

  for b in range(NBUF):
    start_gather(b, b)

  lane128 = lax.iota(jnp.int32, 16) * CHUNK

  def round_body(r, carry):
    for b in range(NBUF):
      g = r * NBUF + b
      pltpu.make_async_copy(table_hbm.at[idx_v.at[g]], rows_v.at[b],
                            in_sems.at[b]).wait()

      @pl.when(g >= NBUF)
      def _wait_out():
        for i in range(8):
          pltpu.make_async_copy(trans_v.at[b, pl.ds(i * 1024, 1024)],
                                out_hbm.at[0, i, 0], out_sems.at[b]).wait()

      def c_body(c, ccarry):
        for m in range(4):
          src = rows_v[b, c, pl.ds(16 * m, 16)]
          plsc.store_scatter(trans_v.at[b],
                             [lane128 + (2048 * m + c)], src)
        return ccarry

      lax.fori_loop(0, CHUNK, c_body, 0)

      c = chunk0 + g
      l = c // n_j
      j = c % n_j
      for i in range(8):
        pltpu.async_copy(trans_v.at[b, pl.ds(i * 1024, 1024)],
                         out_hbm.at[l, i, j], out_sems.at[b])

      @pl.when(g + NBUF < n_chunks)
      def _refill():
        start_gather(g + NBUF, b)

    return carry

  lax.fori_loop(0, n_chunks // NBUF, round_body, 0)

  for b in range(NBUF):
    for i in range(8):
      pltpu.make_async_copy(trans_v.at[b, pl.ds(i * 1024, 1024)],
                            out_hbm.at[0, i, 0], out_sems.at[b]).wait()


def _impl(input_ids, latents, W_emb):
  del latents
  nb, nl = input_ids.shape
  hidden = W_emb.shape[1]
  n_flat = nb * nl
  n_chunks = n_flat // (NW * CHUNK)
  n_j = nb // CHUNK
  ids2d = input_ids.T.reshape(-1, CHUNK).astype(jnp.int32)

  mesh = plsc.VectorSubcoreMesh(core_axis_name="c", subcore_axis_name="s",
                                num_cores=NC, num_subcores=NS)
  fn = pl.kernel(
      functools.partial(_gather_body, n_chunks, n_j),
      out_type=jax.ShapeDtypeStruct((nl, hidden // 8, n_j, 8 * CHUNK),
                                    jnp.float32),
      mesh=mesh,
      scratch_types=[
          pltpu.VMEM((n_chunks, CHUNK), jnp.int32),
          pltpu.VMEM((NBUF, CHUNK, hidden), jnp.float32),
          pltpu.VMEM((NBUF, hidden * CHUNK), jnp.float32),
          pltpu.SemaphoreType.DMA((NBUF,)),
          pltpu.SemaphoreType.DMA((NBUF,)),
      ],
      compiler_params=pltpu.CompilerParams(use_tc_tiling_on_sc=False,
                                           needs_layout_passes=False),
  )
  x4 = fn(ids2d, W_emb)
  x5 = x4.reshape(nl, hidden // 8, n_j, 8, CHUNK)
  return x5.transpose(2, 4, 0, 1, 3).reshape(nb, nl, hidden)


kernel = jax.jit(_impl)
